# Initial kernel scaffold; baseline (speedup 1.0000x reference)
import functools
import jax
import jax.numpy as jnp
from jax import lax
from jax.experimental import pallas as pl
from jax.experimental.pallas import tpu as pltpu
from jax.experimental.pallas import tpu_sc as plsc

N = 10000
E = 320000
D = 128
G = 256

NC = 2
NS = 16
NW = NC * NS
CHUNK = 128
CPT = 80
E_PAD = NW * CPT * CHUNK
NA = 10016
RPT = NA // NS
DUMMY = N + 8


def _sc_aggregate(h, srcs, dsts):
    mesh = plsc.VectorSubcoreMesh(core_axis_name="c", subcore_axis_name="s")

    @functools.partial(
        pl.kernel,
        out_type=jax.ShapeDtypeStruct((NC, NA, D), jnp.float32),
        mesh=mesh,
        scratch_types=[
            pltpu.VMEM((CPT, CHUNK), jnp.int32),
            pltpu.VMEM((CPT, CHUNK), jnp.int32),
            pltpu.VMEM((CHUNK, D), jnp.float32),
            pltpu.VMEM_SHARED((NA, D), jnp.float32),
            pltpu.SemaphoreType.DMA,
        ],
    )
    def k(h_hbm, src_hbm, dst_hbm, out_hbm, src_v, dst_v, rows_v, acc_sh, sem):
        c = lax.axis_index("c")
        s = lax.axis_index("s")
        wid = c * NS + s

        @pl.loop(0, CHUNK)
        def _(i):
            @pl.loop(0, D, step=16)
            def _(j):
                rows_v[i, pl.ds(j, 16)] = jnp.zeros((16,), jnp.float32)

        base = s * RPT
        for t in range(RPT // CHUNK):
            pltpu.sync_copy(rows_v, acc_sh.at[pl.ds(base + t * CHUNK, CHUNK)])
        rem = RPT % CHUNK
        if rem:
            pltpu.sync_copy(rows_v.at[pl.ds(0, rem)],
                            acc_sh.at[pl.ds(base + (RPT // CHUNK) * CHUNK, rem)])

        pltpu.sync_copy(src_hbm.at[pl.ds(wid * CPT, CPT)], src_v)
        pltpu.sync_copy(dst_hbm.at[pl.ds(wid * CPT, CPT)], dst_v)

        plsc.subcore_barrier()

        @pl.loop(0, CPT)
        def _(j):
            pltpu.async_copy(h_hbm.at[src_v.at[j]], rows_v, sem).wait()
            pltpu.sync_copy(rows_v, acc_sh.at[dst_v.at[j]], add=True)

        plsc.subcore_barrier()

        pltpu.sync_copy(acc_sh.at[pl.ds(base, RPT)],
                        out_hbm.at[c].at[pl.ds(base, RPT)])

    return k(h, srcs, dsts)


def _tc_layer(x, aggr, batch2d, W1, b1, W2, b2, gamma, beta):
    def body(x_ref, a_ref, b_ref, w1_ref, b1_ref, w2_ref, b2_ref, g_ref,
             be_ref, h_ref, p_ref):
        t = x_ref[...] + a_ref[0, :N, :] + a_ref[1, :N, :]
        h1 = jnp.dot(t, w1_ref[...], preferred_element_type=jnp.float32,
                     precision=lax.Precision.HIGHEST) + b1_ref[...]
        h1 = jnp.maximum(h1, 0.0)
        u = jnp.dot(h1, w2_ref[...], preferred_element_type=jnp.float32,
                    precision=lax.Precision.HIGHEST) + b2_ref[...]
        u = jnp.maximum(u, 0.0)
        mean = jnp.mean(u, axis=0)
        d = u - mean
        var = jnp.mean(d * d, axis=0)
        h = d * lax.rsqrt(var + 1e-5) * g_ref[...] + be_ref[...]
        h_ref[...] = h
        gids = lax.broadcasted_iota(jnp.int32, (G, N), 0)
        oh = (b_ref[...] == gids).astype(jnp.float32)
        p_ref[...] = jnp.dot(oh, h, preferred_element_type=jnp.float32,
                             precision=lax.Precision.HIGHEST)

    return pl.pallas_call(
        body,
        out_shape=(
            jax.ShapeDtypeStruct((N, D), jnp.float32),
            jax.ShapeDtypeStruct((G, D), jnp.float32),
        ),
        compiler_params=pltpu.CompilerParams(
            vmem_limit_bytes=100 * 1024 * 1024,
        ),
    )(x, aggr, batch2d, W1, b1, W2, b2, gamma, beta)


def kernel(x, edge_index, batch, W1_0, b1_0, W2_0, b2_0, gamma_0, beta_0,
           W1_1, b1_1, W2_1, b2_1, gamma_1, beta_1,
           W1_2, b1_2, W2_2, b2_2, gamma_2, beta_2):
    pad = E_PAD - E
    srcs = jnp.concatenate([edge_index[0], jnp.zeros((pad,), jnp.int32)])
    srcs = srcs.reshape(E_PAD // CHUNK, CHUNK)
    dsts = jnp.concatenate([edge_index[1], jnp.full((pad,), DUMMY, jnp.int32)])
    dsts = dsts.reshape(E_PAD // CHUNK, CHUNK)
    batch2d = batch.reshape(1, N)

    layers = [
        (W1_0, b1_0, W2_0, b2_0, gamma_0, beta_0),
        (W1_1, b1_1, W2_1, b2_1, gamma_1, beta_1),
        (W1_2, b1_2, W2_2, b2_2, gamma_2, beta_2),
    ]
    h = x
    hs, ps = [], []
    for (W1, b1, W2, b2, g, b_) in layers:
        aggr = _sc_aggregate(h, srcs, dsts)
        h, p = _tc_layer(h, aggr, batch2d, W1, b1, W2, b2, g, b_)
        hs.append(h)
        ps.append(p)
    return (jnp.concatenate(ps, axis=1), jnp.concatenate(hs, axis=1))


# speedup vs baseline: 2.6578x; 2.6578x over previous
import functools
import jax
import jax.numpy as jnp
from jax import lax
from jax.experimental import pallas as pl
from jax.experimental.pallas import tpu as pltpu
from jax.experimental.pallas import tpu_sc as plsc

N = 10000
E = 320000
D = 128
G = 256

NC = 2
NS = 16
NW = NC * NS
CHUNK = 128
CPT = 80
E_PAD = NW * CPT * CHUNK
NA = 10112
RPT = NA // NS
DUMMY = N + 8


def _sc_aggregate(h, srcs, dsts):
    mesh = plsc.VectorSubcoreMesh(core_axis_name="c", subcore_axis_name="s")

    @functools.partial(
        pl.kernel,
        out_type=jax.ShapeDtypeStruct((NC, NA, D), jnp.float32),
        mesh=mesh,
        scratch_types=[
            pltpu.VMEM((CPT, CHUNK), jnp.int32),
            pltpu.VMEM((CPT, CHUNK), jnp.int32),
            pltpu.VMEM((CHUNK, D), jnp.float32),
            pltpu.VMEM_SHARED((NA, D), jnp.float32),
            pltpu.SemaphoreType.DMA,
        ],
    )
    def k(h_hbm, src_hbm, dst_hbm, out_hbm, src_v, dst_v, rows_v, acc_sh, sem):
        c = lax.axis_index("c")
        s = lax.axis_index("s")
        wid = c * NS + s

        @pl.loop(0, CHUNK)
        def _(i):
            @pl.loop(0, D, step=16)
            def _(j):
                rows_v[i, pl.ds(j, 16)] = jnp.zeros((16,), jnp.float32)

        base = s * RPT
        for t in range(RPT // CHUNK):
            pltpu.sync_copy(rows_v, acc_sh.at[pl.ds(base + t * CHUNK, CHUNK)])
        rem = RPT % CHUNK
        if rem:
            pltpu.sync_copy(rows_v.at[pl.ds(0, rem)],
                            acc_sh.at[pl.ds(base + (RPT // CHUNK) * CHUNK, rem)])

        pltpu.sync_copy(src_hbm.at[pl.ds(wid * CPT, CPT)], src_v)
        pltpu.sync_copy(dst_hbm.at[pl.ds(wid * CPT, CPT)], dst_v)

        plsc.subcore_barrier()

        @pl.loop(0, CPT)
        def _(j):
            pltpu.async_copy(h_hbm.at[src_v.at[j]], rows_v, sem).wait()
            pltpu.sync_copy(rows_v, acc_sh.at[dst_v.at[j]], add=True)

        plsc.subcore_barrier()

        pltpu.sync_copy(acc_sh.at[pl.ds(base, RPT)],
                        out_hbm.at[c].at[pl.ds(base, RPT)])

    return k(h, srcs, dsts)


def _tc_layer(x, aggr, batch2d, W1, b1, W2, b2, gamma, beta):
    def body(x_ref, a_ref, b_ref, w1_ref, b1_ref, w2_ref, b2_ref, g_ref,
             be_ref, h_ref, p_ref):
        t = x_ref[...] + a_ref[0, :N, :] + a_ref[1, :N, :]
        h1 = jnp.dot(t, w1_ref[...],
                     preferred_element_type=jnp.float32) + b1_ref[...]
        h1 = jnp.maximum(h1, 0.0)
        u = jnp.dot(h1, w2_ref[...],
                    preferred_element_type=jnp.float32) + b2_ref[...]
        u = jnp.maximum(u, 0.0)
        mean = jnp.mean(u, axis=0)
        d = u - mean
        var = jnp.mean(d * d, axis=0)
        h = d * lax.rsqrt(var + 1e-5) * g_ref[...] + be_ref[...]
        h_ref[...] = h
        gids = lax.broadcasted_iota(jnp.int32, (G, N), 0)
        oh = (b_ref[...] == gids).astype(jnp.float32)
        p_ref[...] = jnp.dot(oh, h, preferred_element_type=jnp.float32,
                             precision=lax.Precision.HIGHEST)

    return pl.pallas_call(
        body,
        out_shape=(
            jax.ShapeDtypeStruct((N, D), jnp.float32),
            jax.ShapeDtypeStruct((G, D), jnp.float32),
        ),
        compiler_params=pltpu.CompilerParams(
            vmem_limit_bytes=100 * 1024 * 1024,
        ),
    )(x, aggr, batch2d, W1, b1, W2, b2, gamma, beta)


def kernel(x, edge_index, batch, W1_0, b1_0, W2_0, b2_0, gamma_0, beta_0,
           W1_1, b1_1, W2_1, b2_1, gamma_1, beta_1,
           W1_2, b1_2, W2_2, b2_2, gamma_2, beta_2):
    pad = E_PAD - E
    srcs = jnp.concatenate([edge_index[0], jnp.zeros((pad,), jnp.int32)])
    srcs = srcs.reshape(E_PAD // CHUNK, CHUNK)
    dsts = jnp.concatenate([edge_index[1], jnp.full((pad,), DUMMY, jnp.int32)])
    dsts = dsts.reshape(E_PAD // CHUNK, CHUNK)
    batch2d = batch.reshape(1, N)

    layers = [
        (W1_0, b1_0, W2_0, b2_0, gamma_0, beta_0),
        (W1_1, b1_1, W2_1, b2_1, gamma_1, beta_1),
        (W1_2, b1_2, W2_2, b2_2, gamma_2, beta_2),
    ]
    h = x
    hs, ps = [], []
    for (W1, b1, W2, b2, g, b_) in layers:
        aggr = _sc_aggregate(h, srcs, dsts)
        h, p = _tc_layer(h, aggr, batch2d, W1, b1, W2, b2, g, b_)
        hs.append(h)
        ps.append(p)
    return (jnp.concatenate(ps, axis=1), jnp.concatenate(hs, axis=1))


# speedup vs baseline: 3.7085x; 1.3953x over previous
import functools
import jax
import jax.numpy as jnp
from jax import lax
from jax.experimental import pallas as pl
from jax.experimental.pallas import tpu as pltpu
from jax.experimental.pallas import tpu_sc as plsc

N = 10000
E = 320000
D = 128
G = 256

NC = 2
NS = 16
DH = D // 2
CHUNK = 128
CPT = 160
E_PAD = NS * CPT * CHUNK
NA = 10112
RPT = NA // NS
DUMMY = N + 8
NBUF = 6
LEAD = 3


def _sc_aggregate(h2, srcs2, dsts):
    mesh = plsc.VectorSubcoreMesh(core_axis_name="c", subcore_axis_name="s")

    @functools.partial(
        pl.kernel,
        out_type=jax.ShapeDtypeStruct((NC, NA, DH), jnp.float32),
        mesh=mesh,
        scratch_types=[
            pltpu.VMEM((CPT, CHUNK), jnp.int32),
            pltpu.VMEM((CPT, CHUNK), jnp.int32),
            pltpu.VMEM((NBUF, CHUNK, DH), jnp.float32),
            pltpu.VMEM((8, DH), jnp.float32),
            pltpu.VMEM_SHARED((NA, DH), jnp.float32),
            pltpu.SemaphoreType.DMA((NBUF,)),
            pltpu.SemaphoreType.DMA((NBUF,)),
        ],
        compiler_params=pltpu.CompilerParams(use_tc_tiling_on_sc=False),
    )
    def k(h_hbm, src_hbm, dst_hbm, out_hbm, src_v, dst_v, rows_v, zero_v,
          acc_sh, gsem, ssem):
        c = lax.axis_index("c")
        s = lax.axis_index("s")

        pltpu.sync_copy(src_hbm.at[pl.ds(s * CPT, CPT)], src_v)
        pltpu.sync_copy(dst_hbm.at[pl.ds(s * CPT, CPT)], dst_v)

        @pl.when(c == 1)
        def _():
            @pl.loop(0, CPT)
            def _(i):
                @pl.loop(0, CHUNK, step=16)
                def _(j):
                    src_v[i, pl.ds(j, 16)] = src_v[i, pl.ds(j, 16)] + 1

        @pl.loop(0, 8)
        def _(i):
            @pl.loop(0, DH, step=16)
            def _(j):
                zero_v[i, pl.ds(j, 16)] = jnp.zeros((16,), jnp.float32)

        base = s * RPT

        @pl.loop(0, RPT, step=8)
        def _(r):
            pltpu.sync_copy(zero_v, acc_sh.at[pl.ds(base + r, 8)])

        plsc.subcore_barrier()

        @pl.loop(0, CPT + LEAD)
        def _(t):
            b = lax.rem(t, NBUF)

            @pl.when(t >= NBUF)
            def _():
                j = t - NBUF
                pltpu.make_async_copy(rows_v.at[b], acc_sh.at[dst_v.at[j]],
                                      ssem.at[b]).wait()

            @pl.when(t < CPT)
            def _():
                pltpu.async_copy(h_hbm.at[src_v.at[t]], rows_v.at[b],
                                 gsem.at[b])

            u = t - LEAD

            @pl.when(u >= 0)
            def _():
                bu = lax.rem(u, NBUF)
                pltpu.make_async_copy(h_hbm.at[src_v.at[u]], rows_v.at[bu],
                                      gsem.at[bu]).wait()
                pltpu.async_copy(rows_v.at[bu], acc_sh.at[dst_v.at[u]],
                                 ssem.at[bu], add=True)

        @pl.loop(CPT + LEAD, CPT + NBUF)
        def _(t):
            b = lax.rem(t, NBUF)
            j = t - NBUF
            pltpu.make_async_copy(rows_v.at[b], acc_sh.at[dst_v.at[j]],
                                  ssem.at[b]).wait()

        plsc.subcore_barrier()

        pltpu.sync_copy(acc_sh.at[pl.ds(base, RPT)],
                        out_hbm.at[c].at[pl.ds(base, RPT)])

    return k(h2, srcs2, dsts)


def _tc_layer(x, aggr, batch2d, W1, b1, W2, b2, gamma, beta):
    def body(x_ref, a_ref, b_ref, w1_ref, b1_ref, w2_ref, b2_ref, g_ref,
             be_ref, h_ref, p_ref):
        t = x_ref[...] + jnp.concatenate(
            [a_ref[0, :N, :], a_ref[1, :N, :]], axis=1)
        h1 = jnp.dot(t, w1_ref[...],
                     preferred_element_type=jnp.float32) + b1_ref[...]
        h1 = jnp.maximum(h1, 0.0)
        u = jnp.dot(h1, w2_ref[...],
                    preferred_element_type=jnp.float32) + b2_ref[...]
        u = jnp.maximum(u, 0.0)
        mean = jnp.mean(u, axis=0)
        d = u - mean
        var = jnp.mean(d * d, axis=0)
        h = d * lax.rsqrt(var + 1e-5) * g_ref[...] + be_ref[...]
        h_ref[...] = h
        gids = lax.broadcasted_iota(jnp.int32, (G, N), 0)
        oh = (b_ref[...] == gids).astype(jnp.float32)
        p_ref[...] = jnp.dot(oh, h, preferred_element_type=jnp.float32,
                             precision=lax.Precision.HIGHEST)

    return pl.pallas_call(
        body,
        out_shape=(
            jax.ShapeDtypeStruct((N, D), jnp.float32),
            jax.ShapeDtypeStruct((G, D), jnp.float32),
        ),
        compiler_params=pltpu.CompilerParams(
            vmem_limit_bytes=100 * 1024 * 1024,
        ),
    )(x, aggr, batch2d, W1, b1, W2, b2, gamma, beta)


def kernel(x, edge_index, batch, W1_0, b1_0, W2_0, b2_0, gamma_0, beta_0,
           W1_1, b1_1, W2_1, b2_1, gamma_1, beta_1,
           W1_2, b1_2, W2_2, b2_2, gamma_2, beta_2):
    pad = E_PAD - E
    srcs2 = jnp.concatenate([edge_index[0] * 2, jnp.zeros((pad,), jnp.int32)])
    srcs2 = srcs2.reshape(E_PAD // CHUNK, CHUNK)
    dsts = jnp.concatenate([edge_index[1], jnp.full((pad,), DUMMY, jnp.int32)])
    dsts = dsts.reshape(E_PAD // CHUNK, CHUNK)
    batch2d = batch.reshape(1, N)

    layers = [
        (W1_0, b1_0, W2_0, b2_0, gamma_0, beta_0),
        (W1_1, b1_1, W2_1, b2_1, gamma_1, beta_1),
        (W1_2, b1_2, W2_2, b2_2, gamma_2, beta_2),
    ]
    h = x
    hs, ps = [], []
    for (W1, b1, W2, b2, g, b_) in layers:
        aggr = _sc_aggregate(h.reshape(2 * N, DH), srcs2, dsts)
        h, p = _tc_layer(h, aggr, batch2d, W1, b1, W2, b2, g, b_)
        hs.append(h)
        ps.append(p)
    return (jnp.concatenate(ps, axis=1), jnp.concatenate(hs, axis=1))


# speedup vs baseline: 3.7170x; 1.0023x over previous
import functools
import jax
import jax.numpy as jnp
from jax import lax
from jax.experimental import pallas as pl
from jax.experimental.pallas import tpu as pltpu
from jax.experimental.pallas import tpu_sc as plsc

N = 10000
E = 320000
D = 128
G = 256

NC = 2
NS = 16
DH = D // 2
CHUNK = 128
CPT = 160
E_PAD = NS * CPT * CHUNK
NA = 10112
RPT = NA // NS
DUMMY = N + 8
NBUF = 6
LEAD = 3


def _sc_aggregate(h2, srcs2, dsts, zrows):
    mesh = plsc.VectorSubcoreMesh(core_axis_name="c", subcore_axis_name="s")

    @functools.partial(
        pl.kernel,
        out_type=jax.ShapeDtypeStruct((NC, NA, DH), jnp.float32),
        mesh=mesh,
        scratch_types=[
            pltpu.VMEM((CPT, CHUNK), jnp.int32),
            pltpu.VMEM((CPT, CHUNK), jnp.int32),
            pltpu.VMEM((NBUF, CHUNK, DH), jnp.float32),
            pltpu.VMEM_SHARED((NA, DH), jnp.float32),
            pltpu.SemaphoreType.DMA((NBUF,)),
            pltpu.SemaphoreType.DMA((NBUF,)),
        ],
        compiler_params=pltpu.CompilerParams(use_tc_tiling_on_sc=False),
    )
    def k(h_hbm, src_hbm, dst_hbm, z_hbm, out_hbm, src_v, dst_v, rows_v,
          acc_sh, gsem, ssem):
        c = lax.axis_index("c")
        s = lax.axis_index("s")
        base = s * RPT

        zcp = pltpu.async_copy(z_hbm, acc_sh.at[pl.ds(base, RPT)],
                               gsem.at[0])

        pltpu.sync_copy(src_hbm.at[pl.ds(s * CPT, CPT)], src_v)
        pltpu.sync_copy(dst_hbm.at[pl.ds(s * CPT, CPT)], dst_v)

        @pl.when(c == 1)
        def _():
            @pl.loop(0, CPT)
            def _(i):
                @pl.loop(0, CHUNK, step=16)
                def _(j):
                    src_v[i, pl.ds(j, 16)] = src_v[i, pl.ds(j, 16)] + 1

        zcp.wait()
        plsc.subcore_barrier()

        @pl.loop(0, CPT + LEAD)
        def _(t):
            b = lax.rem(t, NBUF)

            @pl.when(t >= NBUF)
            def _():
                j = t - NBUF
                pltpu.make_async_copy(rows_v.at[b], acc_sh.at[dst_v.at[j]],
                                      ssem.at[b]).wait()

            @pl.when(t < CPT)
            def _():
                pltpu.async_copy(h_hbm.at[src_v.at[t]], rows_v.at[b],
                                 gsem.at[b])

            u = t - LEAD

            @pl.when(u >= 0)
            def _():
                bu = lax.rem(u, NBUF)
                pltpu.make_async_copy(h_hbm.at[src_v.at[u]], rows_v.at[bu],
                                      gsem.at[bu]).wait()
                pltpu.async_copy(rows_v.at[bu], acc_sh.at[dst_v.at[u]],
                                 ssem.at[bu], add=True)

        @pl.loop(CPT + LEAD, CPT + NBUF)
        def _(t):
            b = lax.rem(t, NBUF)
            j = t - NBUF
            pltpu.make_async_copy(rows_v.at[b], acc_sh.at[dst_v.at[j]],
                                  ssem.at[b]).wait()

        plsc.subcore_barrier()

        pltpu.sync_copy(acc_sh.at[pl.ds(base, RPT)],
                        out_hbm.at[c].at[pl.ds(base, RPT)])

    return k(h2, srcs2, dsts, zrows)


def _tc_layer(x, aggr, batch2d, W1, b1, W2, b2, gamma, beta):
    def body(x_ref, a_ref, b_ref, w1_ref, b1_ref, w2_ref, b2_ref, g_ref,
             be_ref, h_ref, p_ref):
        t = x_ref[...] + jnp.concatenate(
            [a_ref[0, :N, :], a_ref[1, :N, :]], axis=1)
        h1 = jnp.dot(t, w1_ref[...],
                     preferred_element_type=jnp.float32) + b1_ref[...]
        h1 = jnp.maximum(h1, 0.0)
        u = jnp.dot(h1, w2_ref[...],
                    preferred_element_type=jnp.float32) + b2_ref[...]
        u = jnp.maximum(u, 0.0)
        mean = jnp.mean(u, axis=0)
        d = u - mean
        var = jnp.mean(d * d, axis=0)
        h = d * lax.rsqrt(var + 1e-5) * g_ref[...] + be_ref[...]
        h_ref[...] = h
        gids = lax.broadcasted_iota(jnp.int32, (G, N), 0)
        oh = (b_ref[...] == gids).astype(jnp.float32)
        p_ref[...] = jnp.dot(oh, h, preferred_element_type=jnp.float32,
                             precision=lax.Precision.HIGHEST)

    return pl.pallas_call(
        body,
        out_shape=(
            jax.ShapeDtypeStruct((N, D), jnp.float32),
            jax.ShapeDtypeStruct((G, D), jnp.float32),
        ),
        compiler_params=pltpu.CompilerParams(
            vmem_limit_bytes=100 * 1024 * 1024,
        ),
    )(x, aggr, batch2d, W1, b1, W2, b2, gamma, beta)


def kernel(x, edge_index, batch, W1_0, b1_0, W2_0, b2_0, gamma_0, beta_0,
           W1_1, b1_1, W2_1, b2_1, gamma_1, beta_1,
           W1_2, b1_2, W2_2, b2_2, gamma_2, beta_2):
    pad = E_PAD - E
    srcs2 = jnp.concatenate([edge_index[0] * 2, jnp.zeros((pad,), jnp.int32)])
    srcs2 = srcs2.reshape(E_PAD // CHUNK, CHUNK)
    dsts = jnp.concatenate([edge_index[1], jnp.full((pad,), DUMMY, jnp.int32)])
    dsts = dsts.reshape(E_PAD // CHUNK, CHUNK)
    batch2d = batch.reshape(1, N)
    zrows = jnp.zeros((RPT, DH), jnp.float32)

    layers = [
        (W1_0, b1_0, W2_0, b2_0, gamma_0, beta_0),
        (W1_1, b1_1, W2_1, b2_1, gamma_1, beta_1),
        (W1_2, b1_2, W2_2, b2_2, gamma_2, beta_2),
    ]
    h = x
    hs, ps = [], []
    for (W1, b1, W2, b2, g, b_) in layers:
        aggr = _sc_aggregate(h.reshape(2 * N, DH), srcs2, dsts, zrows)
        h, p = _tc_layer(h, aggr, batch2d, W1, b1, W2, b2, g, b_)
        hs.append(h)
        ps.append(p)
    return (jnp.concatenate(ps, axis=1), jnp.concatenate(hs, axis=1))


# speedup vs baseline: 3.7235x; 1.0017x over previous
import functools
import jax
import jax.numpy as jnp
from jax import lax
from jax.experimental import pallas as pl
from jax.experimental.pallas import tpu as pltpu
from jax.experimental.pallas import tpu_sc as plsc

N = 10000
E = 320000
D = 128
G = 256

NC = 2
NS = 16
DH = D // 2
CHUNK = 128
CPT = 160
E_PAD = NS * CPT * CHUNK
NA = 10112
RPT = NA // NS
DUMMY = N + 8
NBUF = 6
LEAD = 5


def _sc_aggregate(h2, srcs2, dsts, zrows):
    mesh = plsc.VectorSubcoreMesh(core_axis_name="c", subcore_axis_name="s")

    @functools.partial(
        pl.kernel,
        out_type=jax.ShapeDtypeStruct((NC, NA, DH), jnp.float32),
        mesh=mesh,
        scratch_types=[
            pltpu.VMEM((CPT, CHUNK), jnp.int32),
            pltpu.VMEM((CPT, CHUNK), jnp.int32),
            pltpu.VMEM((NBUF, CHUNK, DH), jnp.float32),
            pltpu.VMEM_SHARED((NA, DH), jnp.float32),
            pltpu.SemaphoreType.DMA((NBUF,)),
            pltpu.SemaphoreType.DMA((NBUF,)),
        ],
        compiler_params=pltpu.CompilerParams(use_tc_tiling_on_sc=False),
    )
    def k(h_hbm, src_hbm, dst_hbm, z_hbm, out_hbm, src_v, dst_v, rows_v,
          acc_sh, gsem, ssem):
        c = lax.axis_index("c")
        s = lax.axis_index("s")
        base = s * RPT

        zcp = pltpu.async_copy(z_hbm, acc_sh.at[pl.ds(base, RPT)],
                               gsem.at[0])

        pltpu.sync_copy(src_hbm.at[pl.ds(s * CPT, CPT)], src_v)
        pltpu.sync_copy(dst_hbm.at[pl.ds(s * CPT, CPT)], dst_v)

        @pl.when(c == 1)
        def _():
            @pl.loop(0, CPT)
            def _(i):
                @pl.loop(0, CHUNK, step=16)
                def _(j):
                    src_v[i, pl.ds(j, 16)] = src_v[i, pl.ds(j, 16)] + 1

        zcp.wait()
        plsc.subcore_barrier()

        @pl.loop(0, CPT + LEAD)
        def _(t):
            b = lax.rem(t, NBUF)

            @pl.when(t >= NBUF)
            def _():
                j = t - NBUF
                pltpu.make_async_copy(rows_v.at[b], acc_sh.at[dst_v.at[j]],
                                      ssem.at[b]).wait()

            @pl.when(t < CPT)
            def _():
                pltpu.async_copy(h_hbm.at[src_v.at[t]], rows_v.at[b],
                                 gsem.at[b])

            u = t - LEAD

            @pl.when(u >= 0)
            def _():
                bu = lax.rem(u, NBUF)
                pltpu.make_async_copy(h_hbm.at[src_v.at[u]], rows_v.at[bu],
                                      gsem.at[bu]).wait()
                pltpu.async_copy(rows_v.at[bu], acc_sh.at[dst_v.at[u]],
                                 ssem.at[bu], add=True)

        @pl.loop(CPT + LEAD, CPT + NBUF)
        def _(t):
            b = lax.rem(t, NBUF)
            j = t - NBUF
            pltpu.make_async_copy(rows_v.at[b], acc_sh.at[dst_v.at[j]],
                                  ssem.at[b]).wait()

        plsc.subcore_barrier()

        pltpu.sync_copy(acc_sh.at[pl.ds(base, RPT)],
                        out_hbm.at[c].at[pl.ds(base, RPT)])

    return k(h2, srcs2, dsts, zrows)


def _tc_layer(x, aggr, batch2d, W1, b1, W2, b2, gamma, beta):
    def body(x_ref, a_ref, b_ref, w1_ref, b1_ref, w2_ref, b2_ref, g_ref,
             be_ref, h_ref, p_ref):
        t = x_ref[...] + jnp.concatenate(
            [a_ref[0, :N, :], a_ref[1, :N, :]], axis=1)
        h1 = jnp.dot(t, w1_ref[...],
                     preferred_element_type=jnp.float32) + b1_ref[...]
        h1 = jnp.maximum(h1, 0.0)
        u = jnp.dot(h1, w2_ref[...],
                    preferred_element_type=jnp.float32) + b2_ref[...]
        u = jnp.maximum(u, 0.0)
        mean = jnp.mean(u, axis=0)
        d = u - mean
        var = jnp.mean(d * d, axis=0)
        h = d * lax.rsqrt(var + 1e-5) * g_ref[...] + be_ref[...]
        h_ref[...] = h
        gids = lax.broadcasted_iota(jnp.int32, (G, N), 0)
        oh = (b_ref[...] == gids).astype(jnp.float32)
        p_ref[...] = jnp.dot(oh, h, preferred_element_type=jnp.float32,
                             precision=lax.Precision.HIGHEST)

    return pl.pallas_call(
        body,
        out_shape=(
            jax.ShapeDtypeStruct((N, D), jnp.float32),
            jax.ShapeDtypeStruct((G, D), jnp.float32),
        ),
        compiler_params=pltpu.CompilerParams(
            vmem_limit_bytes=100 * 1024 * 1024,
        ),
    )(x, aggr, batch2d, W1, b1, W2, b2, gamma, beta)


def kernel(x, edge_index, batch, W1_0, b1_0, W2_0, b2_0, gamma_0, beta_0,
           W1_1, b1_1, W2_1, b2_1, gamma_1, beta_1,
           W1_2, b1_2, W2_2, b2_2, gamma_2, beta_2):
    pad = E_PAD - E
    srcs2 = jnp.concatenate([edge_index[0] * 2, jnp.zeros((pad,), jnp.int32)])
    srcs2 = srcs2.reshape(E_PAD // CHUNK, CHUNK)
    dsts = jnp.concatenate([edge_index[1], jnp.full((pad,), DUMMY, jnp.int32)])
    dsts = dsts.reshape(E_PAD // CHUNK, CHUNK)
    batch2d = batch.reshape(1, N)
    zrows = jnp.zeros((RPT, DH), jnp.float32)

    layers = [
        (W1_0, b1_0, W2_0, b2_0, gamma_0, beta_0),
        (W1_1, b1_1, W2_1, b2_1, gamma_1, beta_1),
        (W1_2, b1_2, W2_2, b2_2, gamma_2, beta_2),
    ]
    h = x
    hs, ps = [], []
    for (W1, b1, W2, b2, g, b_) in layers:
        aggr = _sc_aggregate(h.reshape(2 * N, DH), srcs2, dsts, zrows)
        h, p = _tc_layer(h, aggr, batch2d, W1, b1, W2, b2, g, b_)
        hs.append(h)
        ps.append(p)
    return (jnp.concatenate(ps, axis=1), jnp.concatenate(hs, axis=1))
